# baseline (speedup 1.0000x reference)
import functools

import jax
import jax.numpy as jnp
from jax import lax
from jax.experimental import pallas as pl
from jax.experimental.pallas import tpu as pltpu
from jax.experimental.pallas import tpu_sc as plsc

FEAT = 128
N_EDGES = 320000
NP = 10240
CH = 80
SG = CH // 16

_info = plsc.get_sparse_core_info()
NC = _info.num_cores
NS = _info.num_subcores
NW = NC * NS
EPW = N_EDGES // NW
CHUNKS = EPW // CH
BLK = 25
NBLK = CHUNKS // BLK
RPT = NP // NS

_MESH = plsc.VectorSubcoreMesh(core_axis_name="c", subcore_axis_name="s")


@functools.partial(
    pl.kernel,
    mesh=_MESH,
    out_type=jax.ShapeDtypeStruct((NC, NP), jnp.float32),
    scratch_types=[
        pltpu.VMEM((CHUNKS, CH), jnp.int32),
        pltpu.VMEM((CHUNKS, CH), jnp.float32),
        pltpu.VMEM((RPT,), jnp.float32),
        pltpu.VMEM_SHARED((NP,), jnp.float32),
    ],
)
def _deg_kernel(col_hbm, ew_hbm, out_hbm, col_v, ew_v, zbuf, deg_sh):
    c = lax.axis_index("c")
    s = lax.axis_index("s")
    wid = s * NC + c
    zero16 = jnp.zeros((16,), jnp.float32)

    def zinit(j, _):
        zbuf[pl.ds(j * 16, 16)] = zero16
        return 0

    lax.fori_loop(0, RPT // 16, zinit, 0)
    pltpu.sync_copy(zbuf, deg_sh.at[pl.ds(s * RPT, RPT)])
    pltpu.sync_copy(col_hbm.at[wid], col_v)
    pltpu.sync_copy(ew_hbm.at[wid], ew_v)
    plsc.subcore_barrier()

    def chunk(i, _):
        pltpu.sync_copy(ew_v.at[i], deg_sh.at[col_v.at[i]], add=True)
        return 0

    lax.fori_loop(0, CHUNKS, chunk, 0)
    plsc.subcore_barrier()
    pltpu.sync_copy(deg_sh.at[pl.ds(s * RPT, RPT)],
                    out_hbm.at[c, pl.ds(s * RPT, RPT)])


@functools.partial(
    pl.kernel,
    mesh=_MESH,
    out_type=jax.ShapeDtypeStruct((NC, NP, FEAT), jnp.float32),
    compiler_params=pltpu.CompilerParams(use_tc_tiling_on_sc=False,
                                         needs_layout_passes=False),
    scratch_types=[
        pltpu.VMEM((125, 16), jnp.int32),
        pltpu.VMEM((125, 16), jnp.int32),
        pltpu.VMEM((125, 16), jnp.float32),
        pltpu.VMEM((16, FEAT // 2), jnp.float32),
        pltpu.VMEM((16, FEAT // 2), jnp.float32),
        pltpu.VMEM((16, FEAT), jnp.float32),
        pltpu.VMEM((16, FEAT), jnp.float32),
        pltpu.VMEM_SHARED((NP, FEAT), jnp.float32),
        pltpu.SemaphoreType.DMA,
        pltpu.SemaphoreType.DMA,
        pltpu.SemaphoreType.DMA,
        pltpu.SemaphoreType.DMA,
    ],
)
def _spmm_kernel(gp_hbm, row_hbm, col_hbm, ew_hbm, out_hbm,
                 row_v, col_v, ew_v, ga, gb, sa, sb, acc_sh,
                 gsem0, gsem1, ssem0, ssem1):
    c = lax.axis_index("c")
    s = lax.axis_index("s")
    wid = s * NC + c
    zero16 = jnp.zeros((16,), jnp.float32)
    zidx = jnp.zeros((16,), jnp.int32)
    GB = 125
    NGB = 5

    for j in range(16):
        for d in range(FEAT // 16):
            sa[j, pl.ds(d * 16, 16)] = zero16
    for k in range(RPT // 16):
        pltpu.sync_copy(sa, acc_sh.at[pl.ds(s * RPT + k * 16, 16)])
    plsc.subcore_barrier()

    def scale_group(gbuf, sbuf, q):
        wv = ew_v[q, pl.ds(0, 16)]
        cv = col_v[q, pl.ds(0, 16)]
        for l in range(16):
            w = wv[l]
            for W in range(FEAT // 32):
                u = gbuf[l, pl.ds(W * 16, 16)]
                lo, hi = plsc.unpack(plsc.bitcast(u, jnp.bfloat16),
                                     format=plsc.PackFormat.INTERLEAVED)
                sbuf[l, pl.ds(W * 32, 16)] = lo * w
                sbuf[l, pl.ds(W * 32 + 16, 16)] = hi * w
        return cv

    def gwait(gbuf, gsem):
        pltpu.make_async_copy(gp_hbm.at[row_v.at[0]], gbuf, gsem).wait()

    def sdrain(sbuf, ssem):
        pltpu.make_async_copy(sbuf, acc_sh.at[zidx], ssem).wait()

    def blk(bi, _):
        pltpu.sync_copy(row_hbm.at[wid, bi], row_v)
        pltpu.sync_copy(col_hbm.at[wid, bi], col_v)
        pltpu.sync_copy(ew_hbm.at[wid, bi], ew_v)
        pltpu.async_copy(gp_hbm.at[row_v.at[0]], ga, gsem0)
        pltpu.async_copy(gp_hbm.at[row_v.at[1]], gb, gsem1)

        def pairk(k, _):
            q0 = 2 * k
            q1 = 2 * k + 1
            gwait(ga, gsem0)

            @pl.when(k > 0)
            def _():
                sdrain(sa, ssem0)

            cva = scale_group(ga, sa, q0)
            pltpu.async_copy(sa, acc_sh.at[cva], ssem0, add=True)
            pltpu.async_copy(gp_hbm.at[row_v.at[q0 + 2]], ga, gsem0)
            gwait(gb, gsem1)

            @pl.when(k > 0)
            def _():
                sdrain(sb, ssem1)

            cvb = scale_group(gb, sb, q1)
            pltpu.async_copy(sb, acc_sh.at[cvb], ssem1, add=True)

            @pl.when(q1 + 2 < GB)
            def _():
                pltpu.async_copy(gp_hbm.at[row_v.at[q1 + 2]], gb, gsem1)

            return 0

        lax.fori_loop(0, GB // 2, pairk, 0)
        gwait(ga, gsem0)
        sdrain(sa, ssem0)
        cvt = scale_group(ga, sa, GB - 1)
        pltpu.async_copy(sa, acc_sh.at[cvt], ssem0, add=True)
        sdrain(sa, ssem0)
        sdrain(sb, ssem1)
        return 0

    lax.fori_loop(0, NGB, blk, 0)
    plsc.subcore_barrier()
    pltpu.sync_copy(acc_sh.at[pl.ds(s * RPT, RPT)],
                    out_hbm.at[c, pl.ds(s * RPT, RPT)])


_GRID = NP // 512


def _bs2(r, c_, im):
    return pl.BlockSpec((r, c_), im)


def _pack_bf16(gv):
    parts = []
    for W in range(4):
        rl = gv[:, W * 32:W * 32 + 16].astype(jnp.bfloat16).astype(jnp.float32)
        rh = gv[:, W * 32 + 16:W * 32 + 32].astype(jnp.bfloat16).astype(jnp.float32)
        ul = lax.bitcast_convert_type(rl, jnp.int32)
        uh = lax.bitcast_convert_type(rh, jnp.int32)
        w = jnp.bitwise_or(
            jnp.bitwise_and(uh, jnp.int32(-65536)),
            jnp.bitwise_and(jnp.right_shift(ul, 16), jnp.int32(0xFFFF)))
        parts.append(lax.bitcast_convert_type(w, jnp.float32))
    return jnp.concatenate(parts, axis=1)


def _prep_body(part_ref, x_ref, w_ref, g_ref, gp_ref, dis_ref):
    deg = part_ref[0, :] + part_ref[1, :] + 1.0
    dis = lax.rsqrt(deg)
    h = jnp.dot(x_ref[...], w_ref[...], preferred_element_type=jnp.float32)
    gv = h * dis[:, None]
    g_ref[...] = gv
    gp_ref[...] = _pack_bf16(gv)
    dis_ref[...] = dis[:, None]


def _tc_prep(parts, x, W1):
    return pl.pallas_call(
        _prep_body,
        grid=(_GRID,),
        in_specs=[
            _bs2(2, 512, lambda i: (0, i)),
            _bs2(512, FEAT, lambda i: (i, 0)),
            _bs2(FEAT, FEAT, lambda i: (0, 0)),
        ],
        out_specs=[
            _bs2(512, FEAT, lambda i: (i, 0)),
            _bs2(512, FEAT // 2, lambda i: (i, 0)),
            _bs2(512, 1, lambda i: (i, 0)),
        ],
        out_shape=[
            jax.ShapeDtypeStruct((NP, FEAT), jnp.float32),
            jax.ShapeDtypeStruct((NP, FEAT // 2), jnp.float32),
            jax.ShapeDtypeStruct((NP, 1), jnp.float32),
        ],
    )(parts, x, W1)


def _mid_body(sp_ref, g_ref, dis_ref, b1_ref, w_ref, x_ref, gn_ref,
              gpn_ref):
    sacc = sp_ref[0] + sp_ref[1] + g_ref[...]
    xl = jnp.maximum(dis_ref[...] * sacc + b1_ref[...], 0.0)
    x_ref[...] = xl
    hn = jnp.dot(xl, w_ref[...], preferred_element_type=jnp.float32)
    gv = dis_ref[...] * hn
    gn_ref[...] = gv
    gpn_ref[...] = _pack_bf16(gv)


def _tc_mid(sp, g, dis, b1r, W1):
    return pl.pallas_call(
        _mid_body,
        grid=(_GRID,),
        in_specs=[
            pl.BlockSpec((2, 512, FEAT), lambda i: (0, i, 0)),
            _bs2(512, FEAT, lambda i: (i, 0)),
            _bs2(512, 1, lambda i: (i, 0)),
            _bs2(1, FEAT, lambda i: (0, 0)),
            _bs2(FEAT, FEAT, lambda i: (0, 0)),
        ],
        out_specs=[
            _bs2(512, FEAT, lambda i: (i, 0)),
            _bs2(512, FEAT, lambda i: (i, 0)),
            _bs2(512, FEAT // 2, lambda i: (i, 0)),
        ],
        out_shape=[
            jax.ShapeDtypeStruct((NP, FEAT), jnp.float32),
            jax.ShapeDtypeStruct((NP, FEAT), jnp.float32),
            jax.ShapeDtypeStruct((NP, FEAT // 2), jnp.float32),
        ],
    )(sp, g, dis, b1r, W1)


def _fin_body(sp_ref, g_ref, dis_ref, b1_ref, x1_ref, x2_ref,
              wl_ref, bl_ref, y_ref):
    sacc = sp_ref[0] + sp_ref[1] + g_ref[...]
    x3 = jnp.maximum(dis_ref[...] * sacc + b1_ref[...], 0.0)
    xs = x1_ref[...] + x2_ref[...] + x3
    y_ref[...] = jnp.tanh(
        jnp.dot(xs, wl_ref[...], preferred_element_type=jnp.float32)
        + bl_ref[...])


def _tc_fin(sp, g, dis, b1r, x1, x2, Wl, blr):
    return pl.pallas_call(
        _fin_body,
        grid=(_GRID,),
        in_specs=[
            pl.BlockSpec((2, 512, FEAT), lambda i: (0, i, 0)),
            _bs2(512, FEAT, lambda i: (i, 0)),
            _bs2(512, 1, lambda i: (i, 0)),
            _bs2(1, FEAT, lambda i: (0, 0)),
            _bs2(512, FEAT, lambda i: (i, 0)),
            _bs2(512, FEAT, lambda i: (i, 0)),
            _bs2(FEAT, FEAT, lambda i: (0, 0)),
            _bs2(1, FEAT, lambda i: (0, 0)),
        ],
        out_specs=_bs2(512, FEAT, lambda i: (i, 0)),
        out_shape=jax.ShapeDtypeStruct((NP, FEAT), jnp.float32),
    )(sp, g, dis, b1r, x1, x2, Wl, blr)


def kernel(utter_hidden, edge_index, edge_weight, posemb, W1, b1, Wl, bl):
    turn, batch, _ = utter_hidden.shape
    n = turn * batch
    x = jnp.transpose(utter_hidden, (1, 0, 2)).reshape(n, -1)
    pe = jnp.tile(posemb[:turn], (batch, 1))
    x = jnp.concatenate([x, pe], axis=1)
    x = jnp.zeros((NP, FEAT), jnp.float32).at[:n].set(x)

    row4 = edge_index[0].reshape(NW, 5, 125, 16)
    col4 = edge_index[1].reshape(NW, 5, 125, 16)
    ew4 = edge_weight.reshape(NW, 5, 125, 16)
    col2 = edge_index[1].reshape(NW, CHUNKS, CH)
    ew2 = edge_weight.reshape(NW, CHUNKS, CH)
    b1r = b1.reshape(1, -1)
    blr = bl.reshape(1, -1)

    parts = _deg_kernel(col2, ew2)
    g1, gp1, dis = _tc_prep(parts, x, W1)
    s1 = _spmm_kernel(gp1, row4, col4, ew4)
    x1, g2, gp2 = _tc_mid(s1, g1, dis, b1r, W1)
    s2 = _spmm_kernel(gp2, row4, col4, ew4)
    x2, g3, gp3 = _tc_mid(s2, g2, dis, b1r, W1)
    s3 = _spmm_kernel(gp3, row4, col4, ew4)
    y = _tc_fin(s3, g3, dis, b1r, x1, x2, Wl, blr)
    return y[:n].reshape(batch, turn, -1)


# speedup vs baseline: 1.5421x; 1.5421x over previous
import functools

import jax
import jax.numpy as jnp
from jax import lax
from jax.experimental import pallas as pl
from jax.experimental.pallas import tpu as pltpu
from jax.experimental.pallas import tpu_sc as plsc

FEAT = 128
N_EDGES = 320000
NP = 10240
CH = 80
SG = CH // 16

_info = plsc.get_sparse_core_info()
NC = _info.num_cores
NS = _info.num_subcores
NW = NC * NS
EPW = N_EDGES // NW
CHUNKS = EPW // CH
BLK = 25
NBLK = CHUNKS // BLK
RPT = NP // NS

_MESH = plsc.VectorSubcoreMesh(core_axis_name="c", subcore_axis_name="s")


@functools.partial(
    pl.kernel,
    mesh=_MESH,
    out_type=jax.ShapeDtypeStruct((NC, NP), jnp.float32),
    scratch_types=[
        pltpu.VMEM((CHUNKS, CH), jnp.int32),
        pltpu.VMEM((CHUNKS, CH), jnp.float32),
        pltpu.VMEM((RPT,), jnp.float32),
        pltpu.VMEM_SHARED((NP,), jnp.float32),
    ],
)
def _deg_kernel(col_hbm, ew_hbm, out_hbm, col_v, ew_v, zbuf, deg_sh):
    c = lax.axis_index("c")
    s = lax.axis_index("s")
    wid = s * NC + c
    zero16 = jnp.zeros((16,), jnp.float32)

    def zinit(j, _):
        zbuf[pl.ds(j * 16, 16)] = zero16
        return 0

    lax.fori_loop(0, RPT // 16, zinit, 0)
    pltpu.sync_copy(zbuf, deg_sh.at[pl.ds(s * RPT, RPT)])
    pltpu.sync_copy(col_hbm.at[wid], col_v)
    pltpu.sync_copy(ew_hbm.at[wid], ew_v)
    plsc.subcore_barrier()

    def chunk(i, _):
        pltpu.sync_copy(ew_v.at[i], deg_sh.at[col_v.at[i]], add=True)
        return 0

    lax.fori_loop(0, CHUNKS, chunk, 0)
    plsc.subcore_barrier()
    pltpu.sync_copy(deg_sh.at[pl.ds(s * RPT, RPT)],
                    out_hbm.at[c, pl.ds(s * RPT, RPT)])


@functools.partial(
    pl.kernel,
    mesh=_MESH,
    out_type=(jax.ShapeDtypeStruct((NC, NP, FEAT // 2), jnp.float32),
              jax.ShapeDtypeStruct((NC, NP, FEAT // 2), jnp.float32)),
    compiler_params=pltpu.CompilerParams(use_tc_tiling_on_sc=False,
                                         needs_layout_passes=False),
    scratch_types=[
        pltpu.VMEM((BLK, CH), jnp.int32),
        pltpu.VMEM((BLK, CH), jnp.int32),
        pltpu.VMEM((BLK, CH), jnp.float32),
        pltpu.VMEM((CH, FEAT // 2), jnp.float32),
        pltpu.VMEM((CH, FEAT // 2), jnp.float32),
        pltpu.VMEM((CH, FEAT // 2), jnp.float32),
        pltpu.VMEM((CH, FEAT // 2), jnp.float32),
        pltpu.VMEM_SHARED((NP, FEAT // 2), jnp.float32),
        pltpu.VMEM_SHARED((NP, FEAT // 2), jnp.float32),
        pltpu.SemaphoreType.DMA,
        pltpu.SemaphoreType.DMA,
        pltpu.SemaphoreType.DMA,
        pltpu.SemaphoreType.DMA,
        pltpu.SemaphoreType.DMA,
        pltpu.SemaphoreType.DMA,
    ],
)
def _spmm_kernel(gp_hbm, row_hbm, col_hbm, ew_hbm, out_lo, out_hi,
                 row_v, col_v, ew_v, ga, gb, ha, hb, acc_lo, acc_hi,
                 sga_lo, sga_hi, sgb_lo, sgb_hi, sems_a, sems_b):
    c = lax.axis_index("c")
    s = lax.axis_index("s")
    wid = s * NC + c
    zero16 = jnp.zeros((16,), jnp.float32)
    zidx = jnp.zeros((16,), jnp.int32)
    HF = FEAT // 2

    def zinit(j, _):
        for d in range(HF // 16):
            ga[j, pl.ds(d * 16, 16)] = zero16
        return 0

    lax.fori_loop(0, CH, zinit, 0)
    for k in range(RPT // CH):
        pltpu.sync_copy(ga, acc_lo.at[pl.ds(s * RPT + k * CH, CH)])
        pltpu.sync_copy(ga, acc_hi.at[pl.ds(s * RPT + k * CH, CH)])
    plsc.subcore_barrier()

    LO = 48

    def grp_body(gbuf, hbuf, ssem, i, gi):
        wv = ew_v[i, pl.ds(gi * 16, 16)]
        cv = col_v[i, pl.ds(gi * 16, 16)]
        base = gi * 16
        for l in range(16):
            w = wv[l]
            j = base + l
            for W in range(FEAT // 32):
                sl = pl.ds(W * 16, 16)
                u = gbuf[j, sl]
                lo, hi = plsc.unpack(plsc.bitcast(u, jnp.bfloat16),
                                     format=plsc.PackFormat.INTERLEAVED)
                gbuf[j, sl] = lo * w
                hbuf[j, sl] = hi * w
        pltpu.async_copy(gbuf.at[pl.ds(base, 16)],
                         acc_lo.at[cv], ssem, add=True)
        pltpu.async_copy(hbuf.at[pl.ds(base, 16)],
                         acc_hi.at[cv], ssem, add=True)

    def do_lo(gbuf, hbuf, ssem, i):
        def grp(gi, _):
            grp_body(gbuf, hbuf, ssem, i, gi)
            return 0
        lax.fori_loop(0, LO // 16, grp, 0)

    def do_hi(gbuf, hbuf, ssem, i):
        def grp(gi, _):
            grp_body(gbuf, hbuf, ssem, i, gi)
            return 0
        lax.fori_loop(LO // 16, SG, grp, 0)

    def start_gathers(buf, slo, shi, i):
        pltpu.async_copy(gp_hbm.at[row_v.at[i, pl.ds(0, LO)]],
                         buf.at[pl.ds(0, LO)], slo)
        pltpu.async_copy(gp_hbm.at[row_v.at[i, pl.ds(LO, CH - LO)]],
                         buf.at[pl.ds(LO, CH - LO)], shi)

    def wait_lo(buf, slo, i):
        pltpu.make_async_copy(gp_hbm.at[row_v.at[i, pl.ds(0, LO)]],
                              buf.at[pl.ds(0, LO)], slo).wait()

    def wait_hi(buf, shi, i):
        pltpu.make_async_copy(gp_hbm.at[row_v.at[i, pl.ds(LO, CH - LO)]],
                              buf.at[pl.ds(LO, CH - LO)], shi).wait()

    def drain(gbuf, ssem):
        for _gi in range(2 * SG):
            pltpu.make_async_copy(gbuf.at[pl.ds(0, 16)],
                                  acc_lo.at[zidx], ssem).wait()

    def blk_body(bi, _):
        pltpu.sync_copy(row_hbm.at[wid, bi], row_v)
        pltpu.sync_copy(col_hbm.at[wid, bi], col_v)
        pltpu.sync_copy(ew_hbm.at[wid, bi], ew_v)
        start_gathers(ga, sga_lo, sga_hi, 0)
        start_gathers(gb, sgb_lo, sgb_hi, 1)

        def pair(k, _):
            i0 = 2 * k
            i1 = 2 * k + 1
            i2 = 2 * k + 2
            i3 = 2 * k + 3
            wait_lo(ga, sga_lo, i0)
            do_lo(ga, ha, sems_a, i0)
            wait_hi(ga, sga_hi, i0)
            do_hi(ga, ha, sems_a, i0)
            drain(ga, sems_a)
            start_gathers(ga, sga_lo, sga_hi, i2)
            wait_lo(gb, sgb_lo, i1)
            do_lo(gb, hb, sems_b, i1)
            wait_hi(gb, sgb_hi, i1)
            do_hi(gb, hb, sems_b, i1)
            drain(gb, sems_b)

            @pl.when(i3 < BLK)
            def _():
                start_gathers(gb, sgb_lo, sgb_hi, i3)

            return 0

        lax.fori_loop(0, BLK // 2, pair, 0)
        tail = BLK - 1
        wait_lo(ga, sga_lo, tail)
        do_lo(ga, ha, sems_a, tail)
        wait_hi(ga, sga_hi, tail)
        do_hi(ga, ha, sems_a, tail)
        drain(ga, sems_a)
        return 0

    lax.fori_loop(0, NBLK, blk_body, 0)
    plsc.subcore_barrier()
    pltpu.sync_copy(acc_lo.at[pl.ds(s * RPT, RPT)],
                    out_lo.at[c, pl.ds(s * RPT, RPT)])
    pltpu.sync_copy(acc_hi.at[pl.ds(s * RPT, RPT)],
                    out_hi.at[c, pl.ds(s * RPT, RPT)])


_GRID = NP // 512


def _bs2(r, c_, im):
    return pl.BlockSpec((r, c_), im)


def _pack_bf16(gv):
    parts = []
    for W in range(4):
        rl = gv[:, W * 32:W * 32 + 16].astype(jnp.bfloat16).astype(jnp.float32)
        rh = gv[:, W * 32 + 16:W * 32 + 32].astype(jnp.bfloat16).astype(jnp.float32)
        ul = lax.bitcast_convert_type(rl, jnp.int32)
        uh = lax.bitcast_convert_type(rh, jnp.int32)
        w = jnp.bitwise_or(
            jnp.bitwise_and(uh, jnp.int32(-65536)),
            jnp.bitwise_and(jnp.right_shift(ul, 16), jnp.int32(0xFFFF)))
        parts.append(lax.bitcast_convert_type(w, jnp.float32))
    return jnp.concatenate(parts, axis=1)


def _prep_body(part_ref, x_ref, w_ref, g_ref, gp_ref, dis_ref):
    deg = part_ref[0, :] + part_ref[1, :] + 1.0
    dis = lax.rsqrt(deg)
    h = jnp.dot(x_ref[...], w_ref[...], preferred_element_type=jnp.float32)
    gv = h * dis[:, None]
    g_ref[...] = gv
    gp_ref[...] = _pack_bf16(gv)
    dis_ref[...] = dis[:, None]


def _tc_prep(parts, x, W1):
    return pl.pallas_call(
        _prep_body,
        grid=(_GRID,),
        in_specs=[
            _bs2(2, 512, lambda i: (0, i)),
            _bs2(512, FEAT, lambda i: (i, 0)),
            _bs2(FEAT, FEAT, lambda i: (0, 0)),
        ],
        out_specs=[
            _bs2(512, FEAT, lambda i: (i, 0)),
            _bs2(512, FEAT // 2, lambda i: (i, 0)),
            _bs2(512, 1, lambda i: (i, 0)),
        ],
        out_shape=[
            jax.ShapeDtypeStruct((NP, FEAT), jnp.float32),
            jax.ShapeDtypeStruct((NP, FEAT // 2), jnp.float32),
            jax.ShapeDtypeStruct((NP, 1), jnp.float32),
        ],
    )(parts, x, W1)


def _mid_body(sl_ref, sh_ref, g_ref, dis_ref, b1_ref, w_ref, x_ref, gn_ref,
              gpn_ref):
    slo = sl_ref[0] + sl_ref[1]
    shi = sh_ref[0] + sh_ref[1]
    pieces = []
    for W in range(4):
        pieces.append(slo[:, W * 16:W * 16 + 16])
        pieces.append(shi[:, W * 16:W * 16 + 16])
    sacc = jnp.concatenate(pieces, axis=1) + g_ref[...]
    xl = jnp.maximum(dis_ref[...] * sacc + b1_ref[...], 0.0)
    x_ref[...] = xl
    hn = jnp.dot(xl, w_ref[...], preferred_element_type=jnp.float32)
    gv = dis_ref[...] * hn
    gn_ref[...] = gv
    gpn_ref[...] = _pack_bf16(gv)


def _tc_mid(sl, sh, g, dis, b1r, W1):
    return pl.pallas_call(
        _mid_body,
        grid=(_GRID,),
        in_specs=[
            pl.BlockSpec((2, 512, FEAT // 2), lambda i: (0, i, 0)),
            pl.BlockSpec((2, 512, FEAT // 2), lambda i: (0, i, 0)),
            _bs2(512, FEAT, lambda i: (i, 0)),
            _bs2(512, 1, lambda i: (i, 0)),
            _bs2(1, FEAT, lambda i: (0, 0)),
            _bs2(FEAT, FEAT, lambda i: (0, 0)),
        ],
        out_specs=[
            _bs2(512, FEAT, lambda i: (i, 0)),
            _bs2(512, FEAT, lambda i: (i, 0)),
            _bs2(512, FEAT // 2, lambda i: (i, 0)),
        ],
        out_shape=[
            jax.ShapeDtypeStruct((NP, FEAT), jnp.float32),
            jax.ShapeDtypeStruct((NP, FEAT), jnp.float32),
            jax.ShapeDtypeStruct((NP, FEAT // 2), jnp.float32),
        ],
    )(sl, sh, g, dis, b1r, W1)


def _fin_body(sl_ref, sh_ref, g_ref, dis_ref, b1_ref, x1_ref, x2_ref,
              wl_ref, bl_ref, y_ref):
    slo = sl_ref[0] + sl_ref[1]
    shi = sh_ref[0] + sh_ref[1]
    pieces = []
    for W in range(4):
        pieces.append(slo[:, W * 16:W * 16 + 16])
        pieces.append(shi[:, W * 16:W * 16 + 16])
    sacc = jnp.concatenate(pieces, axis=1) + g_ref[...]
    x3 = jnp.maximum(dis_ref[...] * sacc + b1_ref[...], 0.0)
    xs = x1_ref[...] + x2_ref[...] + x3
    y_ref[...] = jnp.tanh(
        jnp.dot(xs, wl_ref[...], preferred_element_type=jnp.float32)
        + bl_ref[...])


def _tc_fin(sl, sh, g, dis, b1r, x1, x2, Wl, blr):
    return pl.pallas_call(
        _fin_body,
        grid=(_GRID,),
        in_specs=[
            pl.BlockSpec((2, 512, FEAT // 2), lambda i: (0, i, 0)),
            pl.BlockSpec((2, 512, FEAT // 2), lambda i: (0, i, 0)),
            _bs2(512, FEAT, lambda i: (i, 0)),
            _bs2(512, 1, lambda i: (i, 0)),
            _bs2(1, FEAT, lambda i: (0, 0)),
            _bs2(512, FEAT, lambda i: (i, 0)),
            _bs2(512, FEAT, lambda i: (i, 0)),
            _bs2(FEAT, FEAT, lambda i: (0, 0)),
            _bs2(1, FEAT, lambda i: (0, 0)),
        ],
        out_specs=_bs2(512, FEAT, lambda i: (i, 0)),
        out_shape=jax.ShapeDtypeStruct((NP, FEAT), jnp.float32),
    )(sl, sh, g, dis, b1r, x1, x2, Wl, blr)


def kernel(utter_hidden, edge_index, edge_weight, posemb, W1, b1, Wl, bl):
    turn, batch, _ = utter_hidden.shape
    n = turn * batch
    x = jnp.transpose(utter_hidden, (1, 0, 2)).reshape(n, -1)
    pe = jnp.tile(posemb[:turn], (batch, 1))
    x = jnp.concatenate([x, pe], axis=1)
    x = jnp.zeros((NP, FEAT), jnp.float32).at[:n].set(x)

    row4 = edge_index[0].reshape(NW, NBLK, BLK, CH)
    col4 = edge_index[1].reshape(NW, NBLK, BLK, CH)
    ew4 = edge_weight.reshape(NW, NBLK, BLK, CH)
    col2 = edge_index[1].reshape(NW, CHUNKS, CH)
    ew2 = edge_weight.reshape(NW, CHUNKS, CH)
    b1r = b1.reshape(1, -1)
    blr = bl.reshape(1, -1)

    parts = _deg_kernel(col2, ew2)
    g1, gp1, dis = _tc_prep(parts, x, W1)
    s1l, s1h = _spmm_kernel(gp1, row4, col4, ew4)
    x1, g2, gp2 = _tc_mid(s1l, s1h, g1, dis, b1r, W1)
    s2l, s2h = _spmm_kernel(gp2, row4, col4, ew4)
    x2, g3, gp3 = _tc_mid(s2l, s2h, g2, dis, b1r, W1)
    s3l, s3h = _spmm_kernel(gp3, row4, col4, ew4)
    y = _tc_fin(s3l, s3h, g3, dis, b1r, x1, x2, Wl, blr)
    return y[:n].reshape(batch, turn, -1)
